# baseline (speedup 1.0000x reference)
import functools

import jax
import jax.numpy as jnp
from jax import lax
from jax.experimental import pallas as pl
from jax.experimental.pallas import tpu as pltpu
from jax.experimental.pallas import tpu_sc as plsc

_N = 10000
_E = 320000
_D = 128
_DH = 64
_NPAD = 10240
_NSUB = 16
_CH = 128
_NCH = 160
_EPAD = _NSUB * _NCH * _CH
_ZROWS = _NPAD // _NSUB


def _mm1_body(x_ref, w_ref, oa_ref, ob_ref):
    h = jnp.dot(x_ref[...], w_ref[...],
                preferred_element_type=jnp.float32,
                precision=lax.Precision.HIGHEST)
    oa_ref[...] = h[:, :_DH]
    ob_ref[...] = h[:, _DH:]


_matmul1 = pl.pallas_call(
    _mm1_body,
    grid=(5,),
    in_specs=[pl.BlockSpec((2000, _D), lambda i: (i, 0)),
              pl.BlockSpec((_D, _D), lambda i: (0, 0))],
    out_specs=[pl.BlockSpec((2000, _DH), lambda i: (i, 0)),
               pl.BlockSpec((2000, _DH), lambda i: (i, 0))],
    out_shape=(jax.ShapeDtypeStruct((_N, _DH), jnp.float32),
               jax.ShapeDtypeStruct((_N, _DH), jnp.float32)),
)


def _mm2_body(a_ref, b_ref, w_ref, oa_ref, ob_ref):
    h = jnp.maximum(jnp.concatenate([a_ref[...], b_ref[...]], axis=-1), 0.0)
    h = jnp.dot(h, w_ref[...],
                preferred_element_type=jnp.float32,
                precision=lax.Precision.HIGHEST)
    oa_ref[...] = h[:, :_DH]
    ob_ref[...] = h[:, _DH:]


_matmul2 = pl.pallas_call(
    _mm2_body,
    grid=(8,),
    in_specs=[pl.BlockSpec((1280, _DH), lambda i: (i, 0)),
              pl.BlockSpec((1280, _DH), lambda i: (i, 0)),
              pl.BlockSpec((_D, _D), lambda i: (0, 0))],
    out_specs=[pl.BlockSpec((1280, _DH), lambda i: (i, 0)),
               pl.BlockSpec((1280, _DH), lambda i: (i, 0))],
    out_shape=(jax.ShapeDtypeStruct((_NPAD, _DH), jnp.float32),
               jax.ShapeDtypeStruct((_NPAD, _DH), jnp.float32)),
)


def _lsm_body(a_ref, b_ref, o_ref):
    h = jnp.concatenate([a_ref[...], b_ref[...]], axis=-1)
    m = jnp.max(h, axis=-1, keepdims=True)
    e = jnp.exp(h - m)
    s = jnp.sum(e, axis=-1, keepdims=True)
    o_ref[...] = h - m - jnp.log(s)


_logsoftmax = pl.pallas_call(
    _lsm_body,
    grid=(5,),
    in_specs=[pl.BlockSpec((2000, _DH), lambda i: (i, 0)),
              pl.BlockSpec((2000, _DH), lambda i: (i, 0))],
    out_specs=pl.BlockSpec((2000, _D), lambda i: (i, 0)),
    out_shape=jax.ShapeDtypeStruct((_N, _D), jnp.float32),
)


def _make_agg():
    mesh = plsc.VectorSubcoreMesh(core_axis_name="c", subcore_axis_name="s")

    @functools.partial(
        pl.kernel,
        mesh=mesh,
        compiler_params=pltpu.CompilerParams(use_tc_tiling_on_sc=False),
        out_type=(jax.ShapeDtypeStruct((_NPAD, _DH), jnp.float32),
                  jax.ShapeDtypeStruct((_NPAD, _DH), jnp.float32)),
        scratch_types=[
            pltpu.VMEM((_NCH, _CH), jnp.int32),
            pltpu.VMEM((_NCH, _CH), jnp.int32),
            pltpu.VMEM((_CH, _DH), jnp.float32),
            pltpu.VMEM((_CH, _DH), jnp.float32),
            pltpu.VMEM((_CH, _DH), jnp.float32),
            pltpu.VMEM((_CH, _DH), jnp.float32),
            pltpu.VMEM_SHARED((_NPAD, _DH), jnp.float32),
            pltpu.SemaphoreType.DMA,
            pltpu.SemaphoreType.DMA,
            pltpu.SemaphoreType.DMA,
            pltpu.SemaphoreType.DMA,
            pltpu.SemaphoreType.DMA,
            pltpu.SemaphoreType.DMA,
            pltpu.SemaphoreType.DMA,
            pltpu.SemaphoreType.DMA,
        ],
    )
    def agg(ha_hbm, hb_hbm, src_hbm, dst_hbm, zeros_hbm, outa_hbm, outb_hbm,
            src_v, dst_v, buf0, buf1, buf2, buf3, acc_sh,
            gsem0, gsem1, gsem2, gsem3, ssem0, ssem1, ssem2, ssem3):
        c = lax.axis_index("c")
        s = lax.axis_index("s")
        base = s * _NCH
        zbase = s * _ZROWS

        pltpu.sync_copy(src_hbm.at[pl.ds(base, _NCH)], src_v)
        pltpu.sync_copy(dst_hbm.at[pl.ds(base, _NCH)], dst_v)
        pltpu.sync_copy(zeros_hbm, acc_sh.at[pl.ds(zbase, _ZROWS)])
        plsc.subcore_barrier()

        bufs = [buf0, buf1, buf2, buf3]
        gsems = [gsem0, gsem1, gsem2, gsem3]
        ssems = [ssem0, ssem1, ssem2, ssem3]
        nb = 4

        def edge_loop(h_hbm):
            for b in range(nb):
                pltpu.async_copy(h_hbm.at[src_v.at[b]], bufs[b], gsems[b])

            @pl.loop(0, _NCH, step=nb)
            def _edges(i):
                for b in range(nb):
                    c = i + b
                    pltpu.make_async_copy(
                        h_hbm.at[src_v.at[c]], bufs[b], gsems[b]).wait()
                    pltpu.async_copy(
                        bufs[b], acc_sh.at[dst_v.at[c]], ssems[b], add=True)
                    d = (b - 2) % nb
                    cd = c - 2

                    @pl.when(jnp.logical_and(cd >= 0, cd + nb < _NCH))
                    def _():
                        pltpu.make_async_copy(
                            bufs[d], acc_sh.at[dst_v.at[cd]], ssems[d]).wait()
                        pltpu.async_copy(
                            h_hbm.at[src_v.at[cd + nb]], bufs[d], gsems[d])

            for b in range(nb):
                pltpu.make_async_copy(
                    bufs[b], acc_sh.at[dst_v.at[_NCH - nb + b]],
                    ssems[b]).wait()

        @pl.when(c == 0)
        def _():
            edge_loop(ha_hbm)

        @pl.when(c == 1)
        def _():
            edge_loop(hb_hbm)

        plsc.subcore_barrier()

        @pl.when(c == 0)
        def _():
            pltpu.sync_copy(acc_sh.at[pl.ds(zbase, _ZROWS)],
                            outa_hbm.at[pl.ds(zbase, _ZROWS)])

        @pl.when(c == 1)
        def _():
            pltpu.sync_copy(acc_sh.at[pl.ds(zbase, _ZROWS)],
                            outb_hbm.at[pl.ds(zbase, _ZROWS)])

    return agg


_agg = _make_agg()


def kernel(x, edge_index, W1, W2):
    src = edge_index[0].astype(jnp.int32)
    dst = edge_index[1].astype(jnp.int32)
    npad = _EPAD - _E
    src2d = jnp.concatenate(
        [src, jnp.zeros((npad,), jnp.int32)]).reshape(-1, _CH)
    dst2d = jnp.concatenate(
        [dst, jnp.full((npad,), _N, jnp.int32)]).reshape(-1, _CH)
    zeros = jnp.zeros((_ZROWS, _DH), jnp.float32)

    ha, hb = _matmul1(x, W1.T)
    pa, pb = _agg(ha, hb, src2d, dst2d, zeros)
    ga, gb = _matmul2(pa, pb, W2.T)
    qa, qb = _agg(ga, gb, src2d, dst2d, zeros)
    return _logsoftmax(qa, qb)


# speedup vs baseline: 1.2282x; 1.2282x over previous
import functools

import jax
import jax.numpy as jnp
from jax import lax
from jax.experimental import pallas as pl
from jax.experimental.pallas import tpu as pltpu
from jax.experimental.pallas import tpu_sc as plsc

_N = 10000
_E = 320000
_D = 128
_NPAD = 10240
_NCORES = 2
_NSUB = 16
_CH = 128
_NCH = 80
_EPAD = _NCORES * _NSUB * _NCH * _CH
_ZROWS = _NPAD // _NSUB
_NB = 4


def _mm1_body(x_ref, w_ref, o_ref):
    h = jnp.dot(x_ref[...], w_ref[...],
                preferred_element_type=jnp.float32,
                precision=lax.Precision.HIGHEST)
    o_ref[...] = h.astype(jnp.bfloat16)


_matmul1 = pl.pallas_call(
    _mm1_body,
    grid=(5,),
    in_specs=[pl.BlockSpec((2000, _D), lambda i: (i, 0)),
              pl.BlockSpec((_D, _D), lambda i: (0, 0))],
    out_specs=pl.BlockSpec((2000, _D), lambda i: (i, 0)),
    out_shape=jax.ShapeDtypeStruct((_N, _D), jnp.bfloat16),
)


def _mm2_body(a_ref, b_ref, w_ref, o_ref):
    h = a_ref[...].astype(jnp.float32) + b_ref[...].astype(jnp.float32)
    h = jnp.maximum(h, 0.0)
    h = jnp.dot(h, w_ref[...],
                preferred_element_type=jnp.float32,
                precision=lax.Precision.HIGHEST)
    o_ref[...] = h.astype(jnp.bfloat16)


_matmul2 = pl.pallas_call(
    _mm2_body,
    grid=(8,),
    in_specs=[pl.BlockSpec((1280, _D), lambda i: (i, 0)),
              pl.BlockSpec((1280, _D), lambda i: (i, 0)),
              pl.BlockSpec((_D, _D), lambda i: (0, 0))],
    out_specs=pl.BlockSpec((1280, _D), lambda i: (i, 0)),
    out_shape=jax.ShapeDtypeStruct((_NPAD, _D), jnp.bfloat16),
)


def _lsm_body(a_ref, b_ref, o_ref):
    h = a_ref[...].astype(jnp.float32) + b_ref[...].astype(jnp.float32)
    m = jnp.max(h, axis=-1, keepdims=True)
    e = jnp.exp(h - m)
    s = jnp.sum(e, axis=-1, keepdims=True)
    o_ref[...] = h - m - jnp.log(s)


_logsoftmax = pl.pallas_call(
    _lsm_body,
    grid=(5,),
    in_specs=[pl.BlockSpec((2000, _D), lambda i: (i, 0)),
              pl.BlockSpec((2000, _D), lambda i: (i, 0))],
    out_specs=pl.BlockSpec((2000, _D), lambda i: (i, 0)),
    out_shape=jax.ShapeDtypeStruct((_N, _D), jnp.float32),
)


def _make_agg():
    mesh = plsc.VectorSubcoreMesh(core_axis_name="c", subcore_axis_name="s")

    @functools.partial(
        pl.kernel,
        mesh=mesh,
        compiler_params=pltpu.CompilerParams(use_tc_tiling_on_sc=False),
        out_type=(jax.ShapeDtypeStruct((_NPAD, _D), jnp.bfloat16),
                  jax.ShapeDtypeStruct((_NPAD, _D), jnp.bfloat16)),
        scratch_types=[
            pltpu.VMEM((_NCH, _CH), jnp.int32),
            pltpu.VMEM((_NCH, _CH), jnp.int32),
            pltpu.VMEM((_CH, _D), jnp.bfloat16),
            pltpu.VMEM((_CH, _D), jnp.bfloat16),
            pltpu.VMEM((_CH, _D), jnp.bfloat16),
            pltpu.VMEM((_CH, _D), jnp.bfloat16),
            pltpu.VMEM_SHARED((_NPAD, _D), jnp.bfloat16),
            pltpu.SemaphoreType.DMA,
            pltpu.SemaphoreType.DMA,
            pltpu.SemaphoreType.DMA,
            pltpu.SemaphoreType.DMA,
            pltpu.SemaphoreType.DMA,
            pltpu.SemaphoreType.DMA,
            pltpu.SemaphoreType.DMA,
            pltpu.SemaphoreType.DMA,
        ],
    )
    def agg(h_hbm, src_hbm, dst_hbm, zeros_hbm, out0_hbm, out1_hbm,
            src_v, dst_v, buf0, buf1, buf2, buf3, acc_sh,
            gsem0, gsem1, gsem2, gsem3, ssem0, ssem1, ssem2, ssem3):
        c = lax.axis_index("c")
        s = lax.axis_index("s")
        base = (c * _NSUB + s) * _NCH
        zbase = s * _ZROWS

        pltpu.sync_copy(src_hbm.at[pl.ds(base, _NCH)], src_v)
        pltpu.sync_copy(dst_hbm.at[pl.ds(base, _NCH)], dst_v)
        pltpu.sync_copy(zeros_hbm, acc_sh.at[pl.ds(zbase, _ZROWS)])
        plsc.subcore_barrier()

        bufs = [buf0, buf1, buf2, buf3]
        gsems = [gsem0, gsem1, gsem2, gsem3]
        ssems = [ssem0, ssem1, ssem2, ssem3]

        for b in range(_NB):
            pltpu.async_copy(h_hbm.at[src_v.at[b]], bufs[b], gsems[b])

        @pl.loop(0, _NCH, step=_NB)
        def _edges(i):
            for b in range(_NB):
                cc = i + b
                pltpu.make_async_copy(
                    h_hbm.at[src_v.at[cc]], bufs[b], gsems[b]).wait()
                pltpu.async_copy(
                    bufs[b], acc_sh.at[dst_v.at[cc]], ssems[b], add=True)
                d = (b - 2) % _NB
                cd = cc - 2

                @pl.when(jnp.logical_and(cd >= 0, cd + _NB < _NCH))
                def _():
                    pltpu.make_async_copy(
                        bufs[d], acc_sh.at[dst_v.at[cd]], ssems[d]).wait()
                    pltpu.async_copy(
                        h_hbm.at[src_v.at[cd + _NB]], bufs[d], gsems[d])

        for b in range(_NB):
            pltpu.make_async_copy(
                bufs[b], acc_sh.at[dst_v.at[_NCH - _NB + b]], ssems[b]).wait()

        plsc.subcore_barrier()

        @pl.when(c == 0)
        def _():
            pltpu.sync_copy(acc_sh.at[pl.ds(zbase, _ZROWS)],
                            out0_hbm.at[pl.ds(zbase, _ZROWS)])

        @pl.when(c == 1)
        def _():
            pltpu.sync_copy(acc_sh.at[pl.ds(zbase, _ZROWS)],
                            out1_hbm.at[pl.ds(zbase, _ZROWS)])

    return agg


_agg = _make_agg()


def kernel(x, edge_index, W1, W2):
    src = edge_index[0].astype(jnp.int32)
    dst = edge_index[1].astype(jnp.int32)
    npad = _EPAD - _E
    src2d = jnp.concatenate(
        [src, jnp.zeros((npad,), jnp.int32)]).reshape(-1, _CH)
    dst2d = jnp.concatenate(
        [dst, jnp.full((npad,), _N, jnp.int32)]).reshape(-1, _CH)
    zeros = jnp.zeros((_ZROWS, _D), jnp.bfloat16)

    h1 = _matmul1(x, W1.T)
    p0, p1 = _agg(h1, src2d, dst2d, zeros)
    h2 = _matmul2(p0, p1, W2.T)
    q0, q1 = _agg(h2, src2d, dst2d, zeros)
    return _logsoftmax(q0, q1)


# speedup vs baseline: 2.9647x; 2.4139x over previous
import functools

import jax
import jax.numpy as jnp
from jax import lax
from jax.experimental import pallas as pl
from jax.experimental.pallas import tpu as pltpu
from jax.experimental.pallas import tpu_sc as plsc

_N = 10000
_E = 320000
_D = 128
_NPAD = 10240
_NCORES = 2
_NSUB = 16
_CH = 128
_NCH = 80
_EPAD = _NCORES * _NSUB * _NCH * _CH
_ZROWS = _NPAD // _NSUB
_NB = 4


def _mm1_body(x_ref, w_ref, o_ref):
    h = jnp.dot(x_ref[...], w_ref[...],
                preferred_element_type=jnp.float32,
                precision=lax.Precision.HIGHEST)
    o_ref[...] = h.astype(jnp.bfloat16)


_matmul1 = pl.pallas_call(
    _mm1_body,
    grid=(5,),
    in_specs=[pl.BlockSpec((2000, _D), lambda i: (i, 0)),
              pl.BlockSpec((_D, _D), lambda i: (0, 0))],
    out_specs=pl.BlockSpec((2000, _D), lambda i: (i, 0)),
    out_shape=jax.ShapeDtypeStruct((_N, _D), jnp.bfloat16),
)


def _mm2_body(a_ref, b_ref, w_ref, o_ref):
    h = a_ref[...].astype(jnp.float32) + b_ref[...].astype(jnp.float32)
    h = jnp.maximum(h, 0.0)
    h = jnp.dot(h, w_ref[...],
                preferred_element_type=jnp.float32,
                precision=lax.Precision.HIGHEST)
    o_ref[...] = h.astype(jnp.bfloat16)


_matmul2 = pl.pallas_call(
    _mm2_body,
    grid=(8,),
    in_specs=[pl.BlockSpec((1280, _D), lambda i: (i, 0)),
              pl.BlockSpec((1280, _D), lambda i: (i, 0)),
              pl.BlockSpec((_D, _D), lambda i: (0, 0))],
    out_specs=pl.BlockSpec((1280, _D), lambda i: (i, 0)),
    out_shape=jax.ShapeDtypeStruct((_NPAD, _D), jnp.bfloat16),
)


def _lsm_body(a_ref, b_ref, o_ref):
    h = a_ref[...].astype(jnp.float32) + b_ref[...].astype(jnp.float32)
    m = jnp.max(h, axis=-1, keepdims=True)
    e = jnp.exp(h - m)
    s = jnp.sum(e, axis=-1, keepdims=True)
    o_ref[...] = h - m - jnp.log(s)


_logsoftmax = pl.pallas_call(
    _lsm_body,
    grid=(5,),
    in_specs=[pl.BlockSpec((2000, _D), lambda i: (i, 0)),
              pl.BlockSpec((2000, _D), lambda i: (i, 0))],
    out_specs=pl.BlockSpec((2000, _D), lambda i: (i, 0)),
    out_shape=jax.ShapeDtypeStruct((_N, _D), jnp.float32),
)


def _make_agg():
    mesh = plsc.VectorSubcoreMesh(core_axis_name="c", subcore_axis_name="s")

    @functools.partial(
        pl.kernel,
        mesh=mesh,
        compiler_params=pltpu.CompilerParams(use_tc_tiling_on_sc=False),
        out_type=(jax.ShapeDtypeStruct((_NPAD, _D), jnp.bfloat16),
                  jax.ShapeDtypeStruct((_NPAD, _D), jnp.bfloat16)),
        scratch_types=[
            pltpu.VMEM((_NCH, _CH), jnp.int32),
            pltpu.VMEM((_NCH, _CH), jnp.int32),
            pltpu.VMEM((_CH, _D), jnp.bfloat16),
            pltpu.VMEM((_CH, _D), jnp.bfloat16),
            pltpu.VMEM((_CH, _D), jnp.bfloat16),
            pltpu.VMEM((_CH, _D), jnp.bfloat16),
            pltpu.VMEM_SHARED((_NPAD, _D), jnp.bfloat16),
            pltpu.SemaphoreType.DMA,
            pltpu.SemaphoreType.DMA,
            pltpu.SemaphoreType.DMA,
            pltpu.SemaphoreType.DMA,
            pltpu.SemaphoreType.DMA,
            pltpu.SemaphoreType.DMA,
            pltpu.SemaphoreType.DMA,
            pltpu.SemaphoreType.DMA,
        ],
    )
    def agg(h_hbm, src_hbm, dst_hbm, zeros_hbm, out0_hbm, out1_hbm,
            src_v, dst_v, buf0, buf1, buf2, buf3, acc_sh,
            gsem0, gsem1, gsem2, gsem3, ssem0, ssem1, ssem2, ssem3):
        c = lax.axis_index("c")
        s = lax.axis_index("s")
        base = (c * _NSUB + s) * _NCH
        zbase = s * _ZROWS

        pltpu.sync_copy(src_hbm.at[pl.ds(base, _NCH)], src_v)
        pltpu.sync_copy(dst_hbm.at[pl.ds(base, _NCH)], dst_v)
        pltpu.sync_copy(zeros_hbm, acc_sh.at[pl.ds(zbase, _ZROWS)])
        plsc.subcore_barrier()

        bufs = [buf0, buf1, buf2, buf3]
        gsems = [gsem0, gsem1, gsem2, gsem3]
        ssems = [ssem0, ssem1, ssem2, ssem3]

        for b in range(_NB):
            pltpu.async_copy(h_hbm.at[src_v.at[b]], bufs[b], gsems[b])

        @pl.loop(0, _NCH, step=_NB)
        def _edges(i):
            for b in range(_NB):
                cc = i + b
                pltpu.make_async_copy(
                    h_hbm.at[src_v.at[cc]], bufs[b], gsems[b]).wait()
                pltpu.async_copy(
                    bufs[b], acc_sh.at[dst_v.at[cc]], ssems[b], add=True)
                d = (b - 2) % _NB
                cd = cc - 2

                @pl.when(jnp.logical_and(cd >= 0, cd + _NB < _NCH))
                def _():
                    pltpu.make_async_copy(
                        bufs[d], acc_sh.at[dst_v.at[cd]], ssems[d]).wait()
                    pltpu.async_copy(
                        h_hbm.at[src_v.at[cd + _NB]], bufs[d], gsems[d])

        for b in range(_NB):
            pltpu.make_async_copy(
                bufs[b], acc_sh.at[dst_v.at[_NCH - _NB + b]], ssems[b]).wait()

        plsc.subcore_barrier()

        @pl.when(c == 0)
        def _():
            pltpu.sync_copy(acc_sh.at[pl.ds(zbase, _ZROWS)],
                            out0_hbm.at[pl.ds(zbase, _ZROWS)])

        @pl.when(c == 1)
        def _():
            pltpu.sync_copy(acc_sh.at[pl.ds(zbase, _ZROWS)],
                            out1_hbm.at[pl.ds(zbase, _ZROWS)])

    return agg


_agg = _make_agg()


def kernel(x, edge_index, W1, W2):
    src = edge_index[0].astype(jnp.int32)
    dst = edge_index[1].astype(jnp.int32)
    npad = _EPAD - _E
    pad_iota = jnp.arange(npad, dtype=jnp.int32)
    src2d = jnp.concatenate(
        [src, pad_iota % _N]).reshape(-1, _CH)
    dst2d = jnp.concatenate(
        [dst, _N + pad_iota % (_NPAD - _N)]).reshape(-1, _CH)
    zeros = jnp.zeros((_ZROWS, _D), jnp.bfloat16)

    h1 = _matmul1(x, W1.T)
    p0, p1 = _agg(h1, src2d, dst2d, zeros)
    h2 = _matmul2(p0, p1, W2.T)
    q0, q1 = _agg(h2, src2d, dst2d, zeros)
    return _logsoftmax(q0, q1)


# speedup vs baseline: 2.9996x; 1.0118x over previous
import functools

import jax
import jax.numpy as jnp
from jax import lax
from jax.experimental import pallas as pl
from jax.experimental.pallas import tpu as pltpu
from jax.experimental.pallas import tpu_sc as plsc

_N = 10000
_E = 320000
_D = 128
_NCORES = 2
_NSUB = 16
_CH = 125
_NCH = 80
_ZROWS = _N // _NSUB
_NB = 4

_DNUMS = (((1,), (1,)), ((), ()))


def _mm1_body(x_ref, w_ref, o_ref):
    h = lax.dot_general(x_ref[...], w_ref[...], _DNUMS,
                        preferred_element_type=jnp.float32,
                        precision=lax.Precision.HIGHEST)
    o_ref[...] = h.astype(jnp.bfloat16)


_matmul1 = pl.pallas_call(
    _mm1_body,
    grid=(5,),
    in_specs=[pl.BlockSpec((2000, _D), lambda i: (i, 0)),
              pl.BlockSpec((_D, _D), lambda i: (0, 0))],
    out_specs=pl.BlockSpec((2000, _D), lambda i: (i, 0)),
    out_shape=jax.ShapeDtypeStruct((_N, _D), jnp.bfloat16),
)


def _mm2_body(a_ref, b_ref, w_ref, o_ref):
    h = a_ref[...].astype(jnp.float32) + b_ref[...].astype(jnp.float32)
    h = jnp.maximum(h, 0.0)
    h = lax.dot_general(h, w_ref[...], _DNUMS,
                        preferred_element_type=jnp.float32,
                        precision=lax.Precision.HIGHEST)
    o_ref[...] = h.astype(jnp.bfloat16)


_matmul2 = pl.pallas_call(
    _mm2_body,
    grid=(5,),
    in_specs=[pl.BlockSpec((2000, _D), lambda i: (i, 0)),
              pl.BlockSpec((2000, _D), lambda i: (i, 0)),
              pl.BlockSpec((_D, _D), lambda i: (0, 0))],
    out_specs=pl.BlockSpec((2000, _D), lambda i: (i, 0)),
    out_shape=jax.ShapeDtypeStruct((_N, _D), jnp.bfloat16),
)


def _lsm_body(a_ref, b_ref, o_ref):
    h = a_ref[...].astype(jnp.float32) + b_ref[...].astype(jnp.float32)
    m = jnp.max(h, axis=-1, keepdims=True)
    e = jnp.exp(h - m)
    s = jnp.sum(e, axis=-1, keepdims=True)
    o_ref[...] = h - m - jnp.log(s)


_logsoftmax = pl.pallas_call(
    _lsm_body,
    grid=(5,),
    in_specs=[pl.BlockSpec((2000, _D), lambda i: (i, 0)),
              pl.BlockSpec((2000, _D), lambda i: (i, 0))],
    out_specs=pl.BlockSpec((2000, _D), lambda i: (i, 0)),
    out_shape=jax.ShapeDtypeStruct((_N, _D), jnp.float32),
)


def _make_agg():
    mesh = plsc.VectorSubcoreMesh(core_axis_name="c", subcore_axis_name="s")

    @functools.partial(
        pl.kernel,
        mesh=mesh,
        compiler_params=pltpu.CompilerParams(use_tc_tiling_on_sc=False),
        out_type=(jax.ShapeDtypeStruct((_N, _D), jnp.bfloat16),
                  jax.ShapeDtypeStruct((_N, _D), jnp.bfloat16)),
        scratch_types=[
            pltpu.VMEM((_NCH, _CH), jnp.int32),
            pltpu.VMEM((_NCH, _CH), jnp.int32),
            pltpu.VMEM((_CH, _D), jnp.bfloat16),
            pltpu.VMEM((_CH, _D), jnp.bfloat16),
            pltpu.VMEM((_CH, _D), jnp.bfloat16),
            pltpu.VMEM((_CH, _D), jnp.bfloat16),
            pltpu.VMEM_SHARED((_N, _D), jnp.bfloat16),
            pltpu.SemaphoreType.DMA,
            pltpu.SemaphoreType.DMA,
            pltpu.SemaphoreType.DMA,
            pltpu.SemaphoreType.DMA,
            pltpu.SemaphoreType.DMA,
            pltpu.SemaphoreType.DMA,
            pltpu.SemaphoreType.DMA,
            pltpu.SemaphoreType.DMA,
        ],
    )
    def agg(h_hbm, src_hbm, dst_hbm, zeros_hbm, out0_hbm, out1_hbm,
            src_v, dst_v, buf0, buf1, buf2, buf3, acc_sh,
            gsem0, gsem1, gsem2, gsem3, ssem0, ssem1, ssem2, ssem3):
        c = lax.axis_index("c")
        s = lax.axis_index("s")
        base = (c * _NSUB + s) * _NCH
        zbase = s * _ZROWS

        pltpu.sync_copy(src_hbm.at[pl.ds(base, _NCH)], src_v)
        pltpu.sync_copy(dst_hbm.at[pl.ds(base, _NCH)], dst_v)
        pltpu.sync_copy(zeros_hbm, acc_sh.at[pl.ds(zbase, _ZROWS)])
        plsc.subcore_barrier()

        bufs = [buf0, buf1, buf2, buf3]
        gsems = [gsem0, gsem1, gsem2, gsem3]
        ssems = [ssem0, ssem1, ssem2, ssem3]

        for b in range(_NB):
            pltpu.async_copy(h_hbm.at[src_v.at[b]], bufs[b], gsems[b])

        @pl.loop(0, _NCH, step=_NB)
        def _edges(i):
            for b in range(_NB):
                cc = i + b
                pltpu.make_async_copy(
                    h_hbm.at[src_v.at[cc]], bufs[b], gsems[b]).wait()
                pltpu.async_copy(
                    bufs[b], acc_sh.at[dst_v.at[cc]], ssems[b], add=True)
                d = (b - 2) % _NB
                cd = cc - 2

                @pl.when(jnp.logical_and(cd >= 0, cd + _NB < _NCH))
                def _():
                    pltpu.make_async_copy(
                        bufs[d], acc_sh.at[dst_v.at[cd]], ssems[d]).wait()
                    pltpu.async_copy(
                        h_hbm.at[src_v.at[cd + _NB]], bufs[d], gsems[d])

        for b in range(_NB):
            pltpu.make_async_copy(
                bufs[b], acc_sh.at[dst_v.at[_NCH - _NB + b]], ssems[b]).wait()

        plsc.subcore_barrier()

        @pl.when(c == 0)
        def _():
            pltpu.sync_copy(acc_sh.at[pl.ds(zbase, _ZROWS)],
                            out0_hbm.at[pl.ds(zbase, _ZROWS)])

        @pl.when(c == 1)
        def _():
            pltpu.sync_copy(acc_sh.at[pl.ds(zbase, _ZROWS)],
                            out1_hbm.at[pl.ds(zbase, _ZROWS)])

    return agg


_agg = _make_agg()


def kernel(x, edge_index, W1, W2):
    src2d = edge_index[0].astype(jnp.int32).reshape(-1, _CH)
    dst2d = edge_index[1].astype(jnp.int32).reshape(-1, _CH)
    zeros = jnp.zeros((_ZROWS, _D), jnp.bfloat16)

    h1 = _matmul1(x, W1)
    p0, p1 = _agg(h1, src2d, dst2d, zeros)
    h2 = _matmul2(p0, p1, W2)
    q0, q1 = _agg(h2, src2d, dst2d, zeros)
    return _logsoftmax(q0, q1)


# speedup vs baseline: 3.1737x; 1.0581x over previous
import functools

import jax
import jax.numpy as jnp
from jax import lax
from jax.experimental import pallas as pl
from jax.experimental.pallas import tpu as pltpu
from jax.experimental.pallas import tpu_sc as plsc

_N = 10000
_E = 320000
_D = 128
_NCORES = 2
_NSUB = 16
_CH = 125
_NCH = 80
_ZROWS = _N // _NSUB
_NB = 4

_DNUMS = (((1,), (1,)), ((), ()))


def _mm1_body(x_ref, w_ref, o_ref):
    h = lax.dot_general(x_ref[...], w_ref[...], _DNUMS,
                        preferred_element_type=jnp.float32,
                        precision=lax.Precision.HIGHEST)
    o_ref[...] = h.astype(jnp.bfloat16)


_matmul1 = pl.pallas_call(
    _mm1_body,
    grid=(5,),
    in_specs=[pl.BlockSpec((2000, _D), lambda i: (i, 0)),
              pl.BlockSpec((_D, _D), lambda i: (0, 0))],
    out_specs=pl.BlockSpec((2000, _D), lambda i: (i, 0)),
    out_shape=jax.ShapeDtypeStruct((_N, _D), jnp.bfloat16),
)


def _mm2_body(a_ref, b_ref, w_ref, o_ref):
    h = a_ref[...].astype(jnp.float32) + b_ref[...].astype(jnp.float32)
    h = jnp.maximum(h, 0.0)
    h = lax.dot_general(h, w_ref[...], _DNUMS,
                        preferred_element_type=jnp.float32,
                        precision=lax.Precision.HIGHEST)
    o_ref[...] = h.astype(jnp.bfloat16)


_matmul2 = pl.pallas_call(
    _mm2_body,
    grid=(5,),
    in_specs=[pl.BlockSpec((2000, _D), lambda i: (i, 0)),
              pl.BlockSpec((2000, _D), lambda i: (i, 0)),
              pl.BlockSpec((_D, _D), lambda i: (0, 0))],
    out_specs=pl.BlockSpec((2000, _D), lambda i: (i, 0)),
    out_shape=jax.ShapeDtypeStruct((_N, _D), jnp.bfloat16),
)


def _lsm_body(a_ref, b_ref, o_ref):
    h = a_ref[...].astype(jnp.float32) + b_ref[...].astype(jnp.float32)
    m = jnp.max(h, axis=-1, keepdims=True)
    e = jnp.exp(h - m)
    s = jnp.sum(e, axis=-1, keepdims=True)
    o_ref[...] = h - m - jnp.log(s)


_logsoftmax = pl.pallas_call(
    _lsm_body,
    grid=(5,),
    in_specs=[pl.BlockSpec((2000, _D), lambda i: (i, 0)),
              pl.BlockSpec((2000, _D), lambda i: (i, 0))],
    out_specs=pl.BlockSpec((2000, _D), lambda i: (i, 0)),
    out_shape=jax.ShapeDtypeStruct((_N, _D), jnp.float32),
)


def _make_agg():
    mesh = plsc.VectorSubcoreMesh(core_axis_name="c", subcore_axis_name="s")

    @functools.partial(
        pl.kernel,
        mesh=mesh,
        compiler_params=pltpu.CompilerParams(use_tc_tiling_on_sc=False),
        out_type=(jax.ShapeDtypeStruct((_N, _D), jnp.bfloat16),
                  jax.ShapeDtypeStruct((_N, _D), jnp.bfloat16)),
        scratch_types=[
            pltpu.VMEM((_NCH, _CH), jnp.int32),
            pltpu.VMEM((_NCH, _CH), jnp.int32),
            pltpu.VMEM((_CH, _D), jnp.bfloat16),
            pltpu.VMEM((_CH, _D), jnp.bfloat16),
            pltpu.VMEM((_CH, _D), jnp.bfloat16),
            pltpu.VMEM((_CH, _D), jnp.bfloat16),
            pltpu.VMEM_SHARED((_N, _D), jnp.bfloat16),
            pltpu.SemaphoreType.DMA,
            pltpu.SemaphoreType.DMA,
            pltpu.SemaphoreType.DMA,
            pltpu.SemaphoreType.DMA,
            pltpu.SemaphoreType.DMA,
            pltpu.SemaphoreType.DMA,
            pltpu.SemaphoreType.DMA,
            pltpu.SemaphoreType.DMA,
        ],
    )
    def agg(h_hbm, e_hbm, zeros_hbm, out0_hbm, out1_hbm,
            src_v, dst_v, buf0, buf1, buf2, buf3, acc_sh,
            gsem0, gsem1, gsem2, gsem3, ssem0, ssem1, ssem2, ssem3):
        c = lax.axis_index("c")
        s = lax.axis_index("s")
        base = (c * _NSUB + s) * _NCH
        zbase = s * _ZROWS

        pltpu.sync_copy(e_hbm.at[0, pl.ds(base, _NCH)], src_v)
        pltpu.sync_copy(e_hbm.at[1, pl.ds(base, _NCH)], dst_v)
        pltpu.sync_copy(zeros_hbm, acc_sh.at[pl.ds(zbase, _ZROWS)])
        plsc.subcore_barrier()

        bufs = [buf0, buf1, buf2, buf3]
        gsems = [gsem0, gsem1, gsem2, gsem3]
        ssems = [ssem0, ssem1, ssem2, ssem3]

        for b in range(_NB):
            pltpu.async_copy(h_hbm.at[src_v.at[b]], bufs[b], gsems[b])

        @pl.loop(0, _NCH, step=_NB)
        def _edges(i):
            for b in range(_NB):
                cc = i + b
                pltpu.make_async_copy(
                    h_hbm.at[src_v.at[cc]], bufs[b], gsems[b]).wait()
                pltpu.async_copy(
                    bufs[b], acc_sh.at[dst_v.at[cc]], ssems[b], add=True)
                d = (b - 2) % _NB
                cd = cc - 2

                @pl.when(jnp.logical_and(cd >= 0, cd + _NB < _NCH))
                def _():
                    pltpu.make_async_copy(
                        bufs[d], acc_sh.at[dst_v.at[cd]], ssems[d]).wait()
                    pltpu.async_copy(
                        h_hbm.at[src_v.at[cd + _NB]], bufs[d], gsems[d])

        for b in range(_NB):
            pltpu.make_async_copy(
                bufs[b], acc_sh.at[dst_v.at[_NCH - _NB + b]], ssems[b]).wait()

        plsc.subcore_barrier()

        @pl.when(c == 0)
        def _():
            pltpu.sync_copy(acc_sh.at[pl.ds(zbase, _ZROWS)],
                            out0_hbm.at[pl.ds(zbase, _ZROWS)])

        @pl.when(c == 1)
        def _():
            pltpu.sync_copy(acc_sh.at[pl.ds(zbase, _ZROWS)],
                            out1_hbm.at[pl.ds(zbase, _ZROWS)])

    return agg


_agg = _make_agg()


def kernel(x, edge_index, W1, W2):
    e3 = edge_index.astype(jnp.int32).reshape(2, -1, _CH)
    zeros = jnp.zeros((_ZROWS, _D), jnp.bfloat16)

    h1 = _matmul1(x, W1)
    p0, p1 = _agg(h1, e3, zeros)
    h2 = _matmul2(p0, p1, W2)
    q0, q1 = _agg(h2, e3, zeros)
    return _logsoftmax(q0, q1)
